# baseline (speedup 1.0000x reference)
import jax
import jax.numpy as jnp
from jax import lax
from jax.experimental import pallas as pl
from jax.experimental.pallas import tpu as pltpu

H, W, B, NB = 496, 432, 4, 50
TH = 16
NH = H // TH
INV = 1.25
RW = 40


def _boxparams_body(bx_ref, out_ref):
    cx = bx_ref[0, :]
    cy = bx_ref[1, :]
    cz = bx_ref[2, :]
    dx = bx_ref[3, :]
    dy = bx_ref[4, :]
    dz = bx_ref[5, :]
    rz = bx_ref[6, :]
    c = jnp.cos(rz)
    s = jnp.sin(rz)
    zok = jnp.abs(cz) <= dz * 0.5
    cxg = cx * INV
    cyg = cy * INV
    hxg = jnp.where(zok, dx * (0.5 * INV), -1.0)
    hyg = dy * (0.5 * INV)
    r0 = jnp.clip(jnp.floor(cxg) - 20.0, 0.0, float(H - RW))
    j0 = jnp.clip(jnp.floor((cyg - 18.0) / 16.0), 0.0, float((W - 64) // 16))
    out_ref[0, :] = cxg
    out_ref[1, :] = cyg
    out_ref[2, :] = c
    out_ref[3, :] = s
    out_ref[4, :] = hxg
    out_ref[5, :] = hyg
    out_ref[6, :] = r0
    out_ref[7, :] = j0


def _boxparams(boxes):
    bx = boxes.reshape(B * NB, 7).T
    return pl.pallas_call(
        _boxparams_body,
        out_shape=jax.ShapeDtypeStruct((8, B * NB), jnp.float32),
    )(bx)


def _main_body(params_ref, added_ref, orig_ref, iou_ref, inter_s, union_s):
    b = pl.program_id(0)
    h = pl.program_id(1)

    @pl.when(jnp.logical_and(b == 0, h == 0))
    def _():
        iou_ref[0, 0] = 0.0

    @pl.when(h == 0)
    def _():
        for n in range(NB):
            inter_s[0, n] = 0.0
            union_s[0, n] = 0.0

    pred = added_ref[0, 0] + added_ref[0, 1] + added_ref[0, 2] + added_ref[0, 3]
    og = orig_ref[0, 0] + orig_ref[0, 1] + orig_ref[0, 2] + orig_ref[0, 3]
    m_both = jnp.logical_and(pred != 0.0, og != 0.0)
    m_any = jnp.logical_or(pred != 0.0, og != 0.0)

    row0 = (h * TH).astype(jnp.float32)
    ri = lax.broadcasted_iota(jnp.int32, (TH, W), 0).astype(jnp.float32) + row0
    ci = lax.broadcasted_iota(jnp.int32, (TH, W), 1).astype(jnp.float32)
    hth = h * TH

    for n in range(NB):
        idx = b * NB + n
        cxg = params_ref[0, idx]
        cyg = params_ref[1, idx]
        c = params_ref[2, idx]
        s = params_ref[3, idx]
        hxg = params_ref[4, idx]
        hyg = params_ref[5, idx]
        r0i = params_ref[6, idx].astype(jnp.int32)
        overlap = jnp.logical_and(r0i < hth + TH, hth < r0i + RW)

        @pl.when(overlap)
        def _(n=n, cxg=cxg, cyg=cyg, c=c, s=s, hxg=hxg, hyg=hyg):
            u = ri - cxg
            v = ci - cyg
            lx = u * c + v * s
            ly = v * c - u * s
            t = jnp.logical_and(jnp.abs(lx) <= hxg, jnp.abs(ly) <= hyg)
            it = jnp.sum(jnp.where(jnp.logical_and(t, m_both), 1.0, 0.0))
            un = jnp.sum(jnp.where(jnp.logical_and(t, m_any), 1.0, 0.0))
            inter_s[0, n] += it
            union_s[0, n] += un

    @pl.when(h == NH - 1)
    def _():
        tot = jnp.float32(0.0)
        for n in range(NB):
            tot = tot + inter_s[0, n] / jnp.maximum(union_s[0, n], 1.0)
        iou_ref[0, 0] += tot * (1.0 / B)


def kernel(added_points, original_points, boxes):
    params = _boxparams(boxes)
    orig = original_points[:, 1:, :, :]
    out = pl.pallas_call(
        _main_body,
        grid=(B, NH),
        in_specs=[
            pl.BlockSpec(memory_space=pltpu.SMEM),
            pl.BlockSpec((1, 4, TH, W), lambda b, h: (b, 0, h, 0)),
            pl.BlockSpec((1, 4, TH, W), lambda b, h: (b, 0, h, 0)),
        ],
        out_specs=pl.BlockSpec(memory_space=pltpu.SMEM),
        out_shape=jax.ShapeDtypeStruct((1, 1), jnp.float32),
        scratch_shapes=[
            pltpu.SMEM((1, NB), jnp.float32),
            pltpu.SMEM((1, NB), jnp.float32),
        ],
    )(params, added_points, orig)
    return jnp.reshape(out, ())


# speedup vs baseline: 2.6372x; 2.6372x over previous
import functools

import jax
import jax.numpy as jnp
from jax import lax
from jax.experimental import pallas as pl
from jax.experimental.pallas import tpu as pltpu
from jax.experimental.pallas import tpu_sc as plsc

H, W, B, NB = 496, 432, 4, 50
INV = 1.25
RW = 40
WC = W // 16
NBOX = 224
NWORK = 32
BPW = NBOX // NWORK
NROWS = RW * 4


def _boxparams_body(bx_ref, out_ref):
    cx = bx_ref[0, :]
    cy = bx_ref[1, :]
    cz = bx_ref[2, :]
    dx = bx_ref[3, :]
    dy = bx_ref[4, :]
    dz = bx_ref[5, :]
    rz = bx_ref[6, :]
    bidx = bx_ref[7, :]
    c = jnp.cos(rz)
    s = jnp.sin(rz)
    zok = jnp.logical_and(jnp.abs(cz) <= dz * 0.5, bidx >= 0.0)
    cxg = cx * INV
    cyg = cy * INV
    hxg = jnp.where(zok, dx * (0.5 * INV), -1.0)
    hyg = dy * (0.5 * INV)
    r0 = jnp.clip(jnp.floor(cxg) - 20.0, 0.0, float(H - RW))
    j0 = jnp.clip(jnp.floor((cyg - 18.0) / 16.0), 0.0, float(WC - 4))
    bb = jnp.maximum(bidx, 0.0)
    base27j = (bb * float(H) + r0) * float(WC) + j0
    out_ref[0, :] = cxg
    out_ref[1, :] = cyg
    out_ref[2, :] = c
    out_ref[3, :] = s
    out_ref[4, :] = hxg
    out_ref[5, :] = hyg
    out_ref[6, :] = r0
    out_ref[7, :] = j0 * 16.0
    out_ref[8, :] = base27j


def _boxparams(boxes):
    bx = boxes.reshape(B * NB, 7).T
    bxp = jnp.zeros((8, NBOX), jnp.float32)
    bxp = bxp.at[:7, : B * NB].set(bx)
    bidx = jnp.where(
        jnp.arange(NBOX) < B * NB, jnp.arange(NBOX) // NB, -1
    ).astype(jnp.float32)
    bxp = bxp.at[7, :].set(bidx)
    return pl.pallas_call(
        _boxparams_body,
        out_shape=jax.ShapeDtypeStruct((9, NBOX), jnp.float32),
    )(bxp)


TH1 = 248


def _code_body(added_ref, orig_ref, code_ref):
    pred = added_ref[0, 0] + added_ref[0, 1] + added_ref[0, 2] + added_ref[0, 3]
    og = orig_ref[0, 0] + orig_ref[0, 1] + orig_ref[0, 2] + orig_ref[0, 3]
    code_ref[0] = jnp.where(pred != 0.0, 1.0, 0.0) + jnp.where(og != 0.0, 2.0, 0.0)


def _code(added_points, orig):
    return pl.pallas_call(
        _code_body,
        grid=(B, H // TH1),
        in_specs=[
            pl.BlockSpec((1, 4, TH1, W), lambda b, h: (b, 0, h, 0)),
            pl.BlockSpec((1, 4, TH1, W), lambda b, h: (b, 0, h, 0)),
        ],
        out_specs=pl.BlockSpec((1, TH1, W), lambda b, h: (b, h, 0)),
        out_shape=jax.ShapeDtypeStruct((B, H, W), jnp.float32),
    )(added_points, orig)


def _splat_i(val):
    return jnp.full((16,), val, jnp.int32)


def _splat_f(val):
    return jnp.full((16,), val, jnp.float32)


def _sc_body(code_hbm, params_hbm, out_hbm, params_v, idx_a, idx_b, dst,
             outbuf, sem0, sem1):
    wid = lax.axis_index("s") * 2 + lax.axis_index("c")
    pltpu.sync_copy(params_hbm, params_v)
    iota = lax.broadcasted_iota(jnp.int32, (16,), 0)
    iota_f = iota.astype(jnp.float32)
    total_v = jnp.zeros((16,), jnp.float32)

    for k in range(BPW):
        box = wid + NWORK * k
        bsp = _splat_i(box)

        def _p(row):
            return plsc.load_gather(params_v, [_splat_i(row), bsp])

        cxg = _p(0)
        cyg = _p(1)
        c = _p(2)
        s = _p(3)
        hxg = _p(4)
        hyg = _p(5)
        r0f = _p(6)
        c0f = _p(7)
        base_i = _p(8).astype(jnp.int32)

        for t in range(10):
            e = iota + 16 * t
            iv = lax.shift_right_logical(e, 2)
            jj = lax.bitwise_and(e, 3)
            idxv = base_i + iv * WC + jj
            if t < 5:
                idx_a[pl.ds(16 * t, 16)] = idxv
            else:
                idx_b[pl.ds(16 * (t - 5), 16)] = idxv

        cp1 = pltpu.async_copy(code_hbm.at[idx_a], dst.at[pl.ds(0, 80)], sem0)
        cp2 = pltpu.async_copy(code_hbm.at[idx_b], dst.at[pl.ds(80, 80)], sem1)
        cp1.wait()
        cp2.wait()

        u0 = r0f - cxg
        v0 = c0f + iota_f - cyg

        def _chunk_step(it, accs):
            acc_u, acc_i = accs
            di0 = _splat_i(2 * it).astype(jnp.float32)
            for t in range(8):
                ch = it * 8 + t
                u = u0 + (di0 + float(t // 4))
                v = v0 + float(16 * (t % 4))
                lx = u * c + v * s
                ly = v * c - u * s
                m = jnp.logical_and(jnp.abs(lx) <= hxg, jnp.abs(ly) <= hyg)
                codev = plsc.load_gather(dst, [_splat_i(ch), iota])
                acc_u = acc_u + jnp.where(
                    jnp.logical_and(m, codev != 0.0), 1.0, 0.0)
                acc_i = acc_i + jnp.where(
                    jnp.logical_and(m, codev == 3.0), 1.0, 0.0)
            return acc_u, acc_i

        acc_u, acc_i = lax.fori_loop(
            0, NROWS // 8, _chunk_step,
            (jnp.zeros((16,), jnp.float32), jnp.zeros((16,), jnp.float32)))

        us = jnp.sum(acc_u)
        isum = jnp.sum(acc_i)
        total_v = total_v + _splat_f(isum) / jnp.maximum(_splat_f(us), 1.0)

    outbuf[...] = total_v
    pltpu.sync_copy(outbuf, out_hbm.at[wid])


def _sc_counts(code16, params):
    mesh = plsc.VectorSubcoreMesh(core_axis_name="c", subcore_axis_name="s")
    f = functools.partial(
        pl.kernel,
        mesh=mesh,
        compiler_params=pltpu.CompilerParams(
            needs_layout_passes=False, use_tc_tiling_on_sc=False),
        out_type=jax.ShapeDtypeStruct((NWORK, 16), jnp.float32),
        scratch_types=[
            pltpu.VMEM((9, NBOX), jnp.float32),
            pltpu.VMEM((80,), jnp.int32),
            pltpu.VMEM((80,), jnp.int32),
            pltpu.VMEM((NROWS, 16), jnp.float32),
            pltpu.VMEM((16,), jnp.float32),
            pltpu.SemaphoreType.DMA,
            pltpu.SemaphoreType.DMA,
        ],
    )(_sc_body)
    return f(code16, params)


def kernel(added_points, original_points, boxes):
    params = _boxparams(boxes)
    orig = original_points[:, 1:, :, :]
    code = _code(added_points, orig)
    code16 = code.reshape(B * H * WC, 16)
    parts = _sc_counts(code16, params)
    return jnp.sum(parts[:, 0]) * (1.0 / B)
